# Initial kernel scaffold; baseline (speedup 1.0000x reference)
import functools

import jax
import jax.numpy as jnp
from jax import lax
from jax.experimental import pallas as pl
from jax.experimental.pallas import tpu as pltpu
from jax.experimental.pallas import tpu_sc as plsc

_NUM_EMB = 1000000
_D = 32
_B = 16384
_H = 200

_TOT = _B * _H
_NC, _NS = 2, 16
_NW = _NC * _NS
_PER_W = _TOT // _NW
_G = 128
_NG = 8
_BLK = _G * _NG
_NBLK = _PER_W // _BLK


@functools.partial(
    pl.kernel,
    mesh=plsc.VectorSubcoreMesh(core_axis_name="c", subcore_axis_name="s"),
    out_type=jax.ShapeDtypeStruct((_TOT, _D), jnp.float32),
    scratch_types=[
        pltpu.VMEM((_NG, _G), jnp.int32),
        pltpu.VMEM((_BLK, _D), jnp.float32),
        pltpu.SemaphoreType.DMA,
    ],
)
def _emb_gather(idx_hbm, tab_hbm, out_hbm, idx_v, rows_v, sem):
    wid = lax.axis_index("s") * _NC + lax.axis_index("c")
    base = wid * _PER_W

    def body(g, carry):
        start = base + g * _BLK
        pltpu.sync_copy(idx_hbm.at[pl.ds(start, _BLK)].reshape(_NG, _G), idx_v)
        cps = [
            pltpu.async_copy(
                tab_hbm.at[idx_v.at[j]], rows_v.at[pl.ds(j * _G, _G)], sem
            )
            for j in range(_NG)
        ]
        for c in cps:
            c.wait()
        pltpu.sync_copy(rows_v, out_hbm.at[pl.ds(start, _BLK)])
        return carry

    lax.fori_loop(0, _NBLK, body, 0)


def kernel(token_ids, weight):
    flat = token_ids.reshape(_TOT)
    out = _emb_gather(flat, weight)
    return out.reshape(_B, _H, _D)


# speedup vs baseline: 4.8099x; 4.8099x over previous
import functools

import jax
import jax.numpy as jnp
from jax import lax
from jax.experimental import pallas as pl
from jax.experimental.pallas import tpu as pltpu
from jax.experimental.pallas import tpu_sc as plsc

_NUM_EMB = 1000000
_D = 32
_B = 16384
_H = 200

_TOT = _B * _H
_NC, _NS = 2, 16
_NW = _NC * _NS
_PER_W = _TOT // _NW
_G = 128
_NG = 8
_BLK = _G * _NG
_NBLK = _PER_W // _BLK


@functools.partial(
    pl.kernel,
    mesh=plsc.VectorSubcoreMesh(core_axis_name="c", subcore_axis_name="s"),
    out_type=jax.ShapeDtypeStruct((_TOT, _D), jnp.float32),
    compiler_params=pltpu.CompilerParams(use_tc_tiling_on_sc=False),
    scratch_types=[
        pltpu.VMEM((_NG, _G), jnp.int32),
        pltpu.VMEM((_BLK, _D), jnp.float32),
        pltpu.SemaphoreType.DMA,
    ],
)
def _emb_gather(idx_hbm, tab_hbm, out_hbm, idx_v, rows_v, sem):
    wid = lax.axis_index("s") * _NC + lax.axis_index("c")
    base = wid * _PER_W

    def body(g, carry):
        start = base + g * _BLK
        row = pl.multiple_of(start // _G, 8)
        pltpu.sync_copy(idx_hbm.at[pl.ds(row, _NG)], idx_v)
        cps = [
            pltpu.async_copy(
                tab_hbm.at[idx_v.at[j]], rows_v.at[pl.ds(j * _G, _G)], sem
            )
            for j in range(_NG)
        ]
        for c in cps:
            c.wait()
        pltpu.sync_copy(rows_v, out_hbm.at[pl.ds(start, _BLK)])
        return carry

    lax.fori_loop(0, _NBLK, body, 0)


def kernel(token_ids, weight):
    idx2d = token_ids.reshape(_TOT // _G, _G)
    out = _emb_gather(idx2d, weight)
    return out.reshape(_B, _H, _D)
